# baseline (speedup 1.0000x reference)
import jax
import jax.numpy as jnp
from jax import lax
from jax.experimental import pallas as pl
from jax.experimental.pallas import tpu as pltpu
from jax.experimental.pallas import tpu_sc as plsc

NC = 2
NS = 16
NW = NC * NS
L = 16

N = 10000
E = 320000
D_IN = 128
DH = 64

NP = 10240
RB = 5120
NBLK = NP // RB
EPW = E // NW
K = 80
NCHUNK = EPW // K
RPT = NP // NS
NB = 5
NOUT = NCHUNK // NB

_MESH = plsc.VectorSubcoreMesh(core_axis_name="c", subcore_axis_name="s")
_SC_PARAMS = pltpu.CompilerParams(use_tc_tiling_on_sc=False)


def _deg_body(dst3_hbm, out_hbm, deg_sh, didx, ones_v, zer_v, sem):
    cid = lax.axis_index("c")
    sid = lax.axis_index("s")
    wid = sid * NC + cid

    pltpu.sync_copy(dst3_hbm.at[wid], didx)

    def fill_ones(i, _):
        ones_v[pl.ds(i * L, L)] = jnp.full((L,), 1.0, jnp.float32)
        return 0

    lax.fori_loop(0, K // L, fill_ones, 0)

    def fill_zer(i, _):
        zer_v[pl.ds(i * L, L)] = jnp.zeros((L,), jnp.float32)
        return 0

    lax.fori_loop(0, RPT // L, fill_zer, 0)

    pltpu.sync_copy(zer_v, deg_sh.at[pl.ds(sid * RPT, RPT)])
    plsc.subcore_barrier()

    def outer(o, _):
        for j in range(NB):
            pltpu.async_copy(ones_v, deg_sh.at[didx.at[o * NB + j]], sem,
                             add=True)
        for j in range(NB):
            pltpu.make_async_copy(ones_v, deg_sh.at[didx.at[o * NB + j]],
                                  sem).wait()
        return 0

    lax.fori_loop(0, NOUT, outer, 0)
    plsc.subcore_barrier()
    pltpu.sync_copy(deg_sh.at[pl.ds(sid * RPT, RPT)],
                    out_hbm.at[cid, pl.ds(sid * RPT, RPT)])


_deg_call = pl.kernel(
    _deg_body,
    out_type=jax.ShapeDtypeStruct((NC, NP), jnp.float32),
    mesh=_MESH,
    compiler_params=_SC_PARAMS,
    scratch_types=[
        pltpu.VMEM_SHARED((NP,), jnp.float32),
        pltpu.VMEM((NCHUNK, K), jnp.int32),
        pltpu.VMEM((K,), jnp.float32),
        pltpu.VMEM((RPT,), jnp.float32),
        pltpu.SemaphoreType.DMA,
    ],
)


def _agg_body(hp_hbm, src3_hbm, dst3_hbm, zeros_hbm, out_hbm,
              acc_sh, sidx, didxs, rows, gsem, ssem, isem):
    cid = lax.axis_index("c")
    sid = lax.axis_index("s")
    wid = sid * NC + cid

    pltpu.sync_copy(src3_hbm.at[wid], sidx)

    def gather(ci, s):
        pltpu.async_copy(dst3_hbm.at[wid, ci], didxs.at[s], isem.at[s // NB])
        pltpu.async_copy(hp_hbm.at[sidx.at[ci]], rows.at[s],
                         gsem.at[s // NB])

    def wait_gather(ci, s):
        pltpu.make_async_copy(dst3_hbm.at[wid, ci], didxs.at[s],
                              isem.at[s // NB]).wait()
        pltpu.make_async_copy(hp_hbm.at[sidx.at[ci]], rows.at[s],
                              gsem.at[s // NB]).wait()

    def scat(ci, s):
        pltpu.async_copy(rows.at[s], acc_sh.at[didxs.at[s]],
                         ssem.at[s // NB], add=True)

    def wait_scat(ci, s):
        pltpu.make_async_copy(rows.at[s], acc_sh.at[didxs.at[s]],
                              ssem.at[s // NB]).wait()

    for b in range(NB):
        gather(b, b)
    for b in range(NB):
        gather(NB + b, NB + b)

    pltpu.sync_copy(zeros_hbm.at[pl.ds(sid * RPT, RPT)],
                    acc_sh.at[pl.ds(sid * RPT, RPT)])
    plsc.subcore_barrier()

    def outer(oo, _):
        for k in range(3):
            r = oo * 3 + k
            k2 = (k + 2) % 3
            for b in range(NB):
                wait_gather(r * NB + b, k * NB + b)
            for b in range(NB):
                scat(r * NB + b, k * NB + b)
            if k == 0:
                @pl.when(oo > 0)
                def _():
                    for b in range(NB):
                        wait_scat((r - 1) * NB + b, k2 * NB + b)
            else:
                for b in range(NB):
                    wait_scat((r - 1) * NB + b, k2 * NB + b)
            if k == 2:
                @pl.when(oo < (NOUT - 1) // 3 - 1)
                def _():
                    for b in range(NB):
                        gather((r + 2) * NB + b, k2 * NB + b)
            else:
                for b in range(NB):
                    gather((r + 2) * NB + b, k2 * NB + b)
        return 0

    lax.fori_loop(0, (NOUT - 1) // 3, outer, 0)
    for b in range(NB):
        wait_gather((NOUT - 1) * NB + b, b)
    for b in range(NB):
        scat((NOUT - 1) * NB + b, b)
    for b in range(NB):
        wait_scat((NOUT - 2) * NB + b, 2 * NB + b)
    for b in range(NB):
        wait_scat((NOUT - 1) * NB + b, b)
    plsc.subcore_barrier()
    pltpu.sync_copy(acc_sh.at[pl.ds(sid * RPT, RPT)],
                    out_hbm.at[cid, pl.ds(sid * RPT, RPT)])


_agg_call = pl.kernel(
    _agg_body,
    out_type=jax.ShapeDtypeStruct((NC, NP, DH), jnp.float32),
    mesh=_MESH,
    compiler_params=_SC_PARAMS,
    scratch_types=[
        pltpu.VMEM_SHARED((NP, DH), jnp.float32),
        pltpu.VMEM((NCHUNK, K), jnp.int32),
        pltpu.VMEM((3 * NB, K), jnp.int32),
        pltpu.VMEM((3 * NB, K, DH), jnp.float32),
        pltpu.SemaphoreType.DMA((3,)),
        pltpu.SemaphoreType.DMA((3,)),
        pltpu.SemaphoreType.DMA((3,)),
    ],
)


G = RB // 128
EBK = 32000


def _repack_body(ei_ref, src_ref, dst_ref):
    e = ei_ref[...]
    src_ref[...] = e[0]
    dst_ref[...] = e[1]


_repack_call = pl.pallas_call(
    _repack_body,
    out_shape=[jax.ShapeDtypeStruct((E,), jnp.int32),
               jax.ShapeDtypeStruct((E,), jnp.int32)],
)


def _rowscale(deg):
    dinv = lax.rsqrt(deg[0, 0] + deg[1, 0] + 1.0)
    r = lax.broadcasted_iota(jnp.int32, (128, 128), 0)
    c = lax.broadcasted_iota(jnp.int32, (128, 128), 1)
    eye = (r == c).astype(jnp.float32)
    ones = jnp.ones((128, DH), jnp.float32)
    segs = [jnp.dot(jnp.broadcast_to(dinv[j:j + 1, :], (128, 128)) * eye,
                    ones, preferred_element_type=jnp.float32,
                    precision=lax.Precision.HIGHEST)
            for j in range(G)]
    return jnp.concatenate(segs, axis=0)


def _pack(q):
    q3 = jnp.reshape(q, (RB // 2, 2, DH))
    return jnp.concatenate([q3[:, 0, :], q3[:, 1, :]], axis=1)


def _unpack(t):
    s3 = jnp.stack([t[:, :DH], t[:, DH:]], axis=1)
    return jnp.reshape(s3, (RB, DH))


def _mm1_body(x_ref, deg_ref, w1_ref, hp_ref):
    dinv = _rowscale(deg_ref[...])
    h = jnp.dot(x_ref[...], w1_ref[...], preferred_element_type=jnp.float32)
    hp_ref[...] = _pack(dinv * h)


def _mid_body(acc_ref, hp_ref, deg_ref, w2_ref, b1_ref, gp_ref):
    dinv = _rowscale(deg_ref[...])
    acc = acc_ref[...]
    s = _unpack(acc[0] + acc[1] + hp_ref[...])
    z = jnp.maximum(dinv * s + b1_ref[...], 0.0)
    g = jnp.dot(z, w2_ref[...], preferred_element_type=jnp.float32)
    gp_ref[...] = _pack(dinv * g)


def _fin_body(acc_ref, gp_ref, deg_ref, b2_ref, out_ref):
    dinv = _rowscale(deg_ref[...])
    acc = acc_ref[...]
    s = _unpack(acc[0] + acc[1] + gp_ref[...])
    out_ref[...] = dinv * s + b2_ref[...]


_deg_spec = pl.BlockSpec((NC, 1, G, 128), lambda b: (0, b, 0, 0))
_row_spec = pl.BlockSpec((RB, DH), lambda b: (b, 0))
_pack_spec = pl.BlockSpec((RB // 2, 128), lambda b: (b, 0))
_accp_spec = pl.BlockSpec((NC, RB // 2, 128), lambda b: (0, b, 0))
_bias_spec = pl.BlockSpec((1, DH), lambda b: (0, 0))

_mm1_call = pl.pallas_call(
    _mm1_body,
    grid=(NBLK,),
    in_specs=[pl.BlockSpec((RB, D_IN), lambda b: (b, 0)), _deg_spec,
              pl.BlockSpec((D_IN, DH), lambda b: (0, 0))],
    out_specs=_pack_spec,
    out_shape=jax.ShapeDtypeStruct((NP // 2, 128), jnp.float32),
)

_mid_call = pl.pallas_call(
    _mid_body,
    grid=(NBLK,),
    in_specs=[_accp_spec, _pack_spec, _deg_spec,
              pl.BlockSpec((DH, DH), lambda b: (0, 0)), _bias_spec],
    out_specs=_pack_spec,
    out_shape=jax.ShapeDtypeStruct((NP // 2, 128), jnp.float32),
)

_fin_call = pl.pallas_call(
    _fin_body,
    grid=(NBLK,),
    in_specs=[_accp_spec, _pack_spec, _deg_spec, _bias_spec],
    out_specs=_row_spec,
    out_shape=jax.ShapeDtypeStruct((N, DH), jnp.float32),
)


def kernel(x, edge_index, W1, b1, W2, b2):
    src_lin, dst_lin = _repack_call(edge_index)
    src3 = src_lin.reshape(NW, NCHUNK, K)
    dst3 = dst_lin.reshape(NW, NCHUNK, K)

    degp = _deg_call(dst3)
    deg3 = degp.reshape(NC, NBLK, G, 128)

    hp2 = _mm1_call(x, deg3, W1)
    zeros = jnp.zeros((NP, DH), jnp.float32)
    acc1 = _agg_call(hp2.reshape(NP, DH), src3, dst3, zeros)
    acc1p = acc1.reshape(NC, NP // 2, 128)
    gp2 = _mid_call(acc1p, hp2, deg3, W2, b1.reshape(1, DH))
    acc2 = _agg_call(gp2.reshape(NP, DH), src3, dst3, zeros)
    return _fin_call(acc2.reshape(NC, NP // 2, 128), gp2, deg3,
                     b2.reshape(1, DH))


# speedup vs baseline: 1.0033x; 1.0033x over previous
import jax
import jax.numpy as jnp
from jax import lax
from jax.experimental import pallas as pl
from jax.experimental.pallas import tpu as pltpu
from jax.experimental.pallas import tpu_sc as plsc

NC = 2
NS = 16
NW = NC * NS
L = 16

N = 10000
E = 320000
D_IN = 128
DH = 64

NP = 10240
RB = 2048
NBLK = NP // RB
EPW = E // NW
K = 80
NCHUNK = EPW // K
RPT = NP // NS
NB = 5
NOUT = NCHUNK // NB

_MESH = plsc.VectorSubcoreMesh(core_axis_name="c", subcore_axis_name="s")
_SC_PARAMS = pltpu.CompilerParams(use_tc_tiling_on_sc=False)


def _deg_body(dst3_hbm, out_hbm, deg_sh, didx, ones_v, zer_v, sem):
    cid = lax.axis_index("c")
    sid = lax.axis_index("s")
    wid = sid * NC + cid

    pltpu.sync_copy(dst3_hbm.at[wid], didx)

    def fill_ones(i, _):
        ones_v[pl.ds(i * L, L)] = jnp.full((L,), 1.0, jnp.float32)
        return 0

    lax.fori_loop(0, K // L, fill_ones, 0)

    def fill_zer(i, _):
        zer_v[pl.ds(i * L, L)] = jnp.zeros((L,), jnp.float32)
        return 0

    lax.fori_loop(0, RPT // L, fill_zer, 0)

    pltpu.sync_copy(zer_v, deg_sh.at[pl.ds(sid * RPT, RPT)])
    plsc.subcore_barrier()

    def outer(o, _):
        for j in range(NB):
            pltpu.async_copy(ones_v, deg_sh.at[didx.at[o * NB + j]], sem,
                             add=True)
        for j in range(NB):
            pltpu.make_async_copy(ones_v, deg_sh.at[didx.at[o * NB + j]],
                                  sem).wait()
        return 0

    lax.fori_loop(0, NOUT, outer, 0)
    plsc.subcore_barrier()
    pltpu.sync_copy(deg_sh.at[pl.ds(sid * RPT, RPT)],
                    out_hbm.at[cid, pl.ds(sid * RPT, RPT)])


_deg_call = pl.kernel(
    _deg_body,
    out_type=jax.ShapeDtypeStruct((NC, NP), jnp.float32),
    mesh=_MESH,
    compiler_params=_SC_PARAMS,
    scratch_types=[
        pltpu.VMEM_SHARED((NP,), jnp.float32),
        pltpu.VMEM((NCHUNK, K), jnp.int32),
        pltpu.VMEM((K,), jnp.float32),
        pltpu.VMEM((RPT,), jnp.float32),
        pltpu.SemaphoreType.DMA,
    ],
)


def _agg_body(hp_hbm, src3_hbm, dst3_hbm, zeros_hbm, out_hbm,
              acc_sh, sidx, didxs, rows, gsem, ssem, isem):
    cid = lax.axis_index("c")
    sid = lax.axis_index("s")
    wid = sid * NC + cid

    pltpu.sync_copy(src3_hbm.at[wid], sidx)

    def gather(ci, s):
        pltpu.async_copy(dst3_hbm.at[wid, ci], didxs.at[s], isem.at[s // NB])
        pltpu.async_copy(hp_hbm.at[sidx.at[ci]], rows.at[s],
                         gsem.at[s // NB])

    def wait_gather(ci, s):
        pltpu.make_async_copy(dst3_hbm.at[wid, ci], didxs.at[s],
                              isem.at[s // NB]).wait()
        pltpu.make_async_copy(hp_hbm.at[sidx.at[ci]], rows.at[s],
                              gsem.at[s // NB]).wait()

    def scat(ci, s):
        pltpu.async_copy(rows.at[s], acc_sh.at[didxs.at[s]],
                         ssem.at[s // NB], add=True)

    def wait_scat(ci, s):
        pltpu.make_async_copy(rows.at[s], acc_sh.at[didxs.at[s]],
                              ssem.at[s // NB]).wait()

    for b in range(NB):
        gather(b, b)
    for b in range(NB):
        gather(NB + b, NB + b)

    pltpu.sync_copy(zeros_hbm.at[pl.ds(sid * RPT, RPT)],
                    acc_sh.at[pl.ds(sid * RPT, RPT)])
    plsc.subcore_barrier()

    def outer(oo, _):
        for k in range(3):
            r = oo * 3 + k
            k2 = (k + 2) % 3
            for b in range(NB):
                wait_gather(r * NB + b, k * NB + b)
            for b in range(NB):
                scat(r * NB + b, k * NB + b)
            if k == 0:
                @pl.when(oo > 0)
                def _():
                    for b in range(NB):
                        wait_scat((r - 1) * NB + b, k2 * NB + b)
            else:
                for b in range(NB):
                    wait_scat((r - 1) * NB + b, k2 * NB + b)
            if k == 2:
                @pl.when(oo < (NOUT - 1) // 3 - 1)
                def _():
                    for b in range(NB):
                        gather((r + 2) * NB + b, k2 * NB + b)
            else:
                for b in range(NB):
                    gather((r + 2) * NB + b, k2 * NB + b)
        return 0

    lax.fori_loop(0, (NOUT - 1) // 3, outer, 0)
    for b in range(NB):
        wait_gather((NOUT - 1) * NB + b, b)
    for b in range(NB):
        scat((NOUT - 1) * NB + b, b)
    for b in range(NB):
        wait_scat((NOUT - 2) * NB + b, 2 * NB + b)
    for b in range(NB):
        wait_scat((NOUT - 1) * NB + b, b)
    plsc.subcore_barrier()
    pltpu.sync_copy(acc_sh.at[pl.ds(sid * RPT, RPT)],
                    out_hbm.at[cid, pl.ds(sid * RPT, RPT)])


_agg_call = pl.kernel(
    _agg_body,
    out_type=jax.ShapeDtypeStruct((NC, NP, DH), jnp.float32),
    mesh=_MESH,
    compiler_params=_SC_PARAMS,
    scratch_types=[
        pltpu.VMEM_SHARED((NP, DH), jnp.float32),
        pltpu.VMEM((NCHUNK, K), jnp.int32),
        pltpu.VMEM((3 * NB, K), jnp.int32),
        pltpu.VMEM((3 * NB, K, DH), jnp.float32),
        pltpu.SemaphoreType.DMA((3,)),
        pltpu.SemaphoreType.DMA((3,)),
        pltpu.SemaphoreType.DMA((3,)),
    ],
)


G = RB // 128
EBK = 32000


def _repack_body(ei_ref, src_ref, dst_ref):
    e = ei_ref[...]
    src_ref[...] = e[0]
    dst_ref[...] = e[1]


_repack_call = pl.pallas_call(
    _repack_body,
    out_shape=[jax.ShapeDtypeStruct((E,), jnp.int32),
               jax.ShapeDtypeStruct((E,), jnp.int32)],
)


def _rowscale(deg):
    dinv = lax.rsqrt(deg[0, 0] + deg[1, 0] + 1.0)
    r = lax.broadcasted_iota(jnp.int32, (128, 128), 0)
    c = lax.broadcasted_iota(jnp.int32, (128, 128), 1)
    eye = (r == c).astype(jnp.float32)
    ones = jnp.ones((128, DH), jnp.float32)
    segs = [jnp.dot(jnp.broadcast_to(dinv[j:j + 1, :], (128, 128)) * eye,
                    ones, preferred_element_type=jnp.float32,
                    precision=lax.Precision.HIGHEST)
            for j in range(G)]
    return jnp.concatenate(segs, axis=0)


def _pack(q):
    q3 = jnp.reshape(q, (RB // 2, 2, DH))
    return jnp.concatenate([q3[:, 0, :], q3[:, 1, :]], axis=1)


def _unpack(t):
    s3 = jnp.stack([t[:, :DH], t[:, DH:]], axis=1)
    return jnp.reshape(s3, (RB, DH))


def _mm1_body(x_ref, deg_ref, w1_ref, hp_ref):
    dinv = _rowscale(deg_ref[...])
    h = jnp.dot(x_ref[...], w1_ref[...], preferred_element_type=jnp.float32)
    hp_ref[...] = _pack(dinv * h)


def _mid_body(acc_ref, hp_ref, deg_ref, w2_ref, b1_ref, gp_ref):
    dinv = _rowscale(deg_ref[...])
    acc = acc_ref[...]
    s = _unpack(acc[0] + acc[1] + hp_ref[...])
    z = jnp.maximum(dinv * s + b1_ref[...], 0.0)
    g = jnp.dot(z, w2_ref[...], preferred_element_type=jnp.float32)
    gp_ref[...] = _pack(dinv * g)


def _fin_body(acc_ref, gp_ref, deg_ref, b2_ref, out_ref):
    dinv = _rowscale(deg_ref[...])
    acc = acc_ref[...]
    s = _unpack(acc[0] + acc[1] + gp_ref[...])
    out_ref[...] = dinv * s + b2_ref[...]


_deg_spec = pl.BlockSpec((NC, 1, G, 128), lambda b: (0, b, 0, 0))
_row_spec = pl.BlockSpec((RB, DH), lambda b: (b, 0))
_pack_spec = pl.BlockSpec((RB // 2, 128), lambda b: (b, 0))
_accp_spec = pl.BlockSpec((NC, RB // 2, 128), lambda b: (0, b, 0))
_bias_spec = pl.BlockSpec((1, DH), lambda b: (0, 0))

_mm1_call = pl.pallas_call(
    _mm1_body,
    grid=(NBLK,),
    in_specs=[pl.BlockSpec((RB, D_IN), lambda b: (b, 0)), _deg_spec,
              pl.BlockSpec((D_IN, DH), lambda b: (0, 0))],
    out_specs=_pack_spec,
    out_shape=jax.ShapeDtypeStruct((NP // 2, 128), jnp.float32),
)

_mid_call = pl.pallas_call(
    _mid_body,
    grid=(NBLK,),
    in_specs=[_accp_spec, _pack_spec, _deg_spec,
              pl.BlockSpec((DH, DH), lambda b: (0, 0)), _bias_spec],
    out_specs=_pack_spec,
    out_shape=jax.ShapeDtypeStruct((NP // 2, 128), jnp.float32),
)

_fin_call = pl.pallas_call(
    _fin_body,
    grid=(NBLK,),
    in_specs=[_accp_spec, _pack_spec, _deg_spec, _bias_spec],
    out_specs=_row_spec,
    out_shape=jax.ShapeDtypeStruct((N, DH), jnp.float32),
)


def kernel(x, edge_index, W1, b1, W2, b2):
    src_lin, dst_lin = _repack_call(edge_index)
    src3 = src_lin.reshape(NW, NCHUNK, K)
    dst3 = dst_lin.reshape(NW, NCHUNK, K)

    degp = _deg_call(dst3)
    deg3 = degp.reshape(NC, NBLK, G, 128)

    hp2 = _mm1_call(x, deg3, W1)
    zeros = jnp.zeros((NP, DH), jnp.float32)
    acc1 = _agg_call(hp2.reshape(NP, DH), src3, dst3, zeros)
    acc1p = acc1.reshape(NC, NP // 2, 128)
    gp2 = _mid_call(acc1p, hp2, deg3, W2, b1.reshape(1, DH))
    acc2 = _agg_call(gp2.reshape(NP, DH), src3, dst3, zeros)
    return _fin_call(acc2.reshape(NC, NP // 2, 128), gp2, deg3,
                     b2.reshape(1, DH))
